# Initial kernel scaffold; baseline (speedup 1.0000x reference)
import jax
import jax.numpy as jnp
from jax.experimental import pallas as pl


def kernel(input_emb, position, pe):
    raise NotImplementedError("write your pallas kernel here")


# speedup vs baseline: 1.2306x; 1.2306x over previous
import jax
import jax.numpy as jnp
from jax import lax
from jax.experimental import pallas as pl
from jax.experimental.pallas import tpu as pltpu
from jax.experimental.pallas import tpu_sc as plsc

NC = 2
NS = 16
NW = NC * NS

CH = 80


def _make_sc_call(R, D, nch):
    rpw = R // NW
    mesh = plsc.VectorSubcoreMesh(core_axis_name="c", subcore_axis_name="s")

    def body(x_hbm, idx_hbm, pe_hbm, out_hbm, idx_v, buf, sem):
        wid = lax.axis_index("s") * NC + lax.axis_index("c")
        pltpu.sync_copy(idx_hbm.at[wid], idx_v)
        for j in range(nch):
            base = wid * rpw + j * CH
            pltpu.sync_copy(x_hbm.at[pl.ds(base, CH)], buf)
            pltpu.async_copy(pe_hbm.at[idx_v.at[j]], buf, sem, add=True).wait()
            pltpu.sync_copy(buf, out_hbm.at[pl.ds(base, CH)])

    return pl.kernel(
        body,
        out_type=jax.ShapeDtypeStruct((R, D), jnp.float32),
        mesh=mesh,
        scratch_types=[
            pltpu.VMEM((nch, CH), jnp.int32),
            pltpu.VMEM((CH, D), jnp.float32),
            pltpu.SemaphoreType.DMA,
        ],
    )


def kernel(input_emb, position, pe):
    B, N, L, D = input_emb.shape
    R = B * N * L
    rpw = R // NW
    nch = rpw // CH
    x = input_emb.reshape(R, D)
    idx = position.reshape(NW, nch, CH).astype(jnp.int32)
    out = _make_sc_call(R, D, nch)(x, idx, pe)
    return out.reshape(B, N, L, D)


# speedup vs baseline: 1.4142x; 1.1492x over previous
import jax
import jax.numpy as jnp
from jax import lax
from jax.experimental import pallas as pl
from jax.experimental.pallas import tpu as pltpu
from jax.experimental.pallas import tpu_sc as plsc

NC = 2
NS = 16
NW = NC * NS

CH = 80
NBUF = 8
LA = 4
LB = 2


def _make_sc_call(R, D, nch):
    rpw = R // NW
    mesh = plsc.VectorSubcoreMesh(core_axis_name="c", subcore_axis_name="s")

    def body(x_hbm, idx_hbm, pe_hbm, out_hbm, idx_v, bufs, sem_in, sem_g, sem_out):
        wid = lax.axis_index("s") * NC + lax.axis_index("c")
        pltpu.sync_copy(idx_hbm.at[wid], idx_v)

        h_in = [None] * nch
        h_g = [None] * nch
        h_out = [None] * nch
        out_waited = [False] * nch

        def s1(j):
            b = j % NBUF
            h_in[j] = pltpu.async_copy(
                x_hbm.at[pl.ds(wid * rpw + j * CH, CH)], bufs.at[b], sem_in.at[b])

        def s2(j):
            b = j % NBUF
            h_in[j].wait()
            h_g[j] = pltpu.async_copy(
                pe_hbm.at[idx_v.at[j]], bufs.at[b], sem_g.at[b], add=True)

        def s3(j):
            b = j % NBUF
            h_g[j].wait()
            h_out[j] = pltpu.async_copy(
                bufs.at[b], out_hbm.at[pl.ds(wid * rpw + j * CH, CH)], sem_out.at[b])

        for j in range(min(LA, nch)):
            s1(j)
        for j in range(min(LB, nch)):
            s2(j)
        for j in range(nch):
            ja = j + LA
            if ja < nch:
                if ja >= NBUF:
                    h_out[ja - NBUF].wait()
                    out_waited[ja - NBUF] = True
                s1(ja)
            jb = j + LB
            if jb < nch:
                s2(jb)
            s3(j)
        for j in range(nch):
            if not out_waited[j]:
                h_out[j].wait()

    return pl.kernel(
        body,
        out_type=jax.ShapeDtypeStruct((R, D), jnp.float32),
        mesh=mesh,
        scratch_types=[
            pltpu.VMEM((nch, CH), jnp.int32),
            pltpu.VMEM((NBUF, CH, D), jnp.float32),
            pltpu.SemaphoreType.DMA((NBUF,)),
            pltpu.SemaphoreType.DMA((NBUF,)),
            pltpu.SemaphoreType.DMA((NBUF,)),
        ],
    )


def kernel(input_emb, position, pe):
    B, N, L, D = input_emb.shape
    R = B * N * L
    rpw = R // NW
    nch = rpw // CH
    x = input_emb.reshape(R, D)
    idx = position.reshape(NW, nch, CH).astype(jnp.int32)
    out = _make_sc_call(R, D, nch)(x, idx, pe)
    return out.reshape(B, N, L, D)


# speedup vs baseline: 2.0740x; 1.4666x over previous
import jax
import jax.numpy as jnp
from jax import lax
from jax.experimental import pallas as pl
from jax.experimental.pallas import tpu as pltpu
from jax.experimental.pallas import tpu_sc as plsc

NC = 2
NS = 16
NW = NC * NS

NBUF = 8
LA = 4
LB = 2


def _make_sc_call(G, L, D):
    gpw = G // NW
    mesh = plsc.VectorSubcoreMesh(core_axis_name="c", subcore_axis_name="s")

    def body(x_hbm, idx_hbm, pe_hbm, out_hbm, idx_v, bufs, sem_in, sem_g, sem_out):
        wid = lax.axis_index("s") * NC + lax.axis_index("c")
        pltpu.sync_copy(idx_hbm.at[pl.ds(wid * gpw, gpw)], idx_v)

        h_in = [None] * gpw
        h_g = [None] * gpw
        h_out = [None] * gpw
        out_waited = [False] * gpw

        def s1(j):
            h_in[j] = pltpu.async_copy(
                x_hbm.at[wid * gpw + j], bufs.at[j % NBUF], sem_in.at[j % NBUF])

        def s2(j):
            h_in[j].wait()
            h_g[j] = pltpu.async_copy(
                pe_hbm.at[idx_v.at[j]], bufs.at[j % NBUF], sem_g.at[j % NBUF],
                add=True)

        def s3(j):
            h_g[j].wait()
            h_out[j] = pltpu.async_copy(
                bufs.at[j % NBUF], out_hbm.at[wid * gpw + j], sem_out.at[j % NBUF])

        for j in range(min(LA, gpw)):
            s1(j)
        for j in range(min(LB, gpw)):
            s2(j)
        for j in range(gpw):
            ja = j + LA
            if ja < gpw:
                if ja >= NBUF:
                    h_out[ja - NBUF].wait()
                    out_waited[ja - NBUF] = True
                s1(ja)
            jb = j + LB
            if jb < gpw:
                s2(jb)
            s3(j)
        for j in range(gpw):
            if not out_waited[j]:
                h_out[j].wait()

    return pl.kernel(
        body,
        out_type=jax.ShapeDtypeStruct((G, L, D), jnp.float32),
        mesh=mesh,
        scratch_types=[
            pltpu.VMEM((gpw, L), jnp.int32),
            pltpu.VMEM((NBUF, L, D), jnp.float32),
            pltpu.SemaphoreType.DMA((NBUF,)),
            pltpu.SemaphoreType.DMA((NBUF,)),
            pltpu.SemaphoreType.DMA((NBUF,)),
        ],
    )


def kernel(input_emb, position, pe):
    B, N, L, D = input_emb.shape
    G = B * N
    x = input_emb.reshape(G, L, D)
    idx = position.reshape(G, L).astype(jnp.int32)
    out = _make_sc_call(G, L, D)(x, idx, pe)
    return out.reshape(B, N, L, D)
